# baseline (speedup 1.0000x reference)
import functools

import jax
import jax.numpy as jnp
from jax import lax
from jax.experimental import pallas as pl
from jax.experimental.pallas import tpu as pltpu
from jax.experimental.pallas import tpu_sc as plsc

N = 10000
NPAD = 10240
E = 320000
DF = 128
DOUT = 128
GROUP = 5
NGRP = N // GROUP
NC = 2
NS = 16
NW = NC * NS
EPW = E // NW
EPWP = 10240
CH = 128
NCH = EPWP // CH
NBUF = 2
RPW = NPAD // NS
SCHUNK = 128
SPW = 3 * SCHUNK
NSEED = NW * SPW

_R = 2000
_GRID = N // _R


def _sc_agg(tab_hbm, src3_hbm, dst3_hbm, z128_hbm, agg_out,
            srcb, dstb, b0, b1, s0, s1, d0, d1, agg_sh):
    c = lax.axis_index("c")
    s = lax.axis_index("s")
    w = s * NC + c
    r0 = s * RPW
    bufs = (b0, b1)
    sems = (s0, s1)
    dsems = (d0, d1)
    pltpu.sync_copy(z128_hbm.at[pl.ds(r0, RPW)], agg_sh.at[pl.ds(r0, RPW)])
    pltpu.sync_copy(src3_hbm.at[w], srcb)
    plsc.subcore_barrier()

    pltpu.async_copy(tab_hbm.at[srcb.at[0]], bufs[0], sems[0])
    pltpu.async_copy(dst3_hbm.at[w, 0], dstb.at[0], dsems[0])

    def outer(t, carry):
        for b in range(NBUF):
            g = NBUF * t + b
            pltpu.make_async_copy(tab_hbm.at[srcb.at[g]],
                                  bufs[b], sems[b]).wait()
            pltpu.make_async_copy(dst3_hbm.at[w, g],
                                  dstb.at[b], dsems[b]).wait()

            @pl.when(g < NCH - 1)
            def _prefetch():
                nb = 1 - b
                pltpu.async_copy(tab_hbm.at[srcb.at[g + 1]],
                                 bufs[nb], sems[nb])
                pltpu.async_copy(dst3_hbm.at[w, g + 1],
                                 dstb.at[nb], dsems[nb])

            pltpu.sync_copy(bufs[b], agg_sh.at[dstb.at[b]], add=True)
        return carry

    lax.fori_loop(0, NCH // NBUF, outer, 0)
    plsc.subcore_barrier()
    pltpu.sync_copy(agg_sh.at[pl.ds(r0, RPW)],
                    agg_out.at[c, pl.ds(r0, RPW)])


def _sc_seed_gather(out_hbm, seed3_hbm, emb_out, idxb, r0, r1, r2,
                    s0, s1, s2):
    c = lax.axis_index("c")
    s = lax.axis_index("s")
    w = s * NC + c
    rbufs = (r0, r1, r2)
    sems = (s0, s1, s2)
    pltpu.sync_copy(seed3_hbm.at[w], idxb)
    for j in range(3):
        pltpu.async_copy(out_hbm.at[idxb.at[j]], rbufs[j], sems[j])
    for j in range(3):
        pltpu.make_async_copy(out_hbm.at[idxb.at[j]], rbufs[j],
                              sems[j]).wait()
        pltpu.async_copy(rbufs[j],
                         emb_out.at[pl.ds(w * SPW + j * SCHUNK, SCHUNK)],
                         sems[j])
    for j in range(3):
        pltpu.make_async_copy(
            rbufs[j], emb_out.at[pl.ds(w * SPW + j * SCHUNK, SCHUNK)],
            sems[j]).wait()


_sc_calls = None


def _build_sc_calls():
    global _sc_calls
    if _sc_calls is not None:
        return _sc_calls
    mesh = plsc.VectorSubcoreMesh(core_axis_name="c", subcore_axis_name="s")

    def make_agg():
        def body(*args):
            return _sc_agg(*args)
        return pl.kernel(
            body, mesh=mesh,
            out_type=[jax.ShapeDtypeStruct((NC, NPAD, DF), jnp.float32)],
            scratch_types=[
                pltpu.VMEM((NCH, CH), jnp.int32),
                pltpu.VMEM((NBUF, CH), jnp.int32),
                pltpu.VMEM((CH, DF), jnp.float32),
                pltpu.VMEM((CH, DF), jnp.float32),
                pltpu.SemaphoreType.DMA,
                pltpu.SemaphoreType.DMA,
                pltpu.SemaphoreType.DMA,
                pltpu.SemaphoreType.DMA,
                pltpu.VMEM_SHARED((NPAD, DF), jnp.float32),
            ])

    agg_call = make_agg()
    agg2_call = make_agg()
    seed_call = pl.kernel(
        _sc_seed_gather, mesh=mesh,
        out_type=[jax.ShapeDtypeStruct((NSEED, DOUT), jnp.float32)],
        scratch_types=[
            pltpu.VMEM((3, SCHUNK), jnp.int32),
            pltpu.VMEM((SCHUNK, DOUT), jnp.float32),
            pltpu.VMEM((SCHUNK, DOUT), jnp.float32),
            pltpu.VMEM((SCHUNK, DOUT), jnp.float32),
            pltpu.SemaphoreType.DMA,
            pltpu.SemaphoreType.DMA,
            pltpu.SemaphoreType.DMA,
        ])
    _sc_calls = (agg_call, agg2_call, seed_call)
    return _sc_calls


_BE = 6400
_NBE = E // _BE


def _tc_deg(dst_ref, out_ref):
    i = pl.program_id(0)

    @pl.when(i == 0)
    def _init():
        out_ref[...] = jnp.zeros_like(out_ref)

    d = dst_ref[...]
    ih = lax.broadcasted_iota(jnp.int32, (_BE, 128), 1)
    oh_hi = ((d >> 7) == ih).astype(jnp.float32)
    oh_lo = ((d & 127) == ih).astype(jnp.float32)
    out_ref[...] += lax.dot_general(
        oh_hi, oh_lo, (((0,), (0,)), ((), ())),
        preferred_element_type=jnp.float32)

    @pl.when(i == _NBE - 1)
    def _clip():
        out_ref[...] = jnp.maximum(out_ref[...], 1.0)


def _tc_deg_call(dst):
    deg2d = pl.pallas_call(
        _tc_deg,
        grid=(_NBE,),
        in_specs=[pl.BlockSpec((_BE, 1), lambda i: (i, 0))],
        out_specs=pl.BlockSpec((128, 128), lambda i: (0, 0)),
        out_shape=jax.ShapeDtypeStruct((128, 128), jnp.float32),
    )(dst.reshape(E, 1))
    return deg2d.reshape(-1)[:N].reshape(N, 1)


def _tc_layer(x_ref, agga_ref, aggb_ref, deg_ref, w_ref, wn_ref,
              h_ref, *, relu):
    agg = (agga_ref[0] + aggb_ref[0]) / deg_ref[...]
    acc = (jnp.dot(x_ref[...], w_ref[...], preferred_element_type=jnp.float32)
           + jnp.dot(agg, wn_ref[...], preferred_element_type=jnp.float32))
    h_ref[...] = jnp.maximum(acc, 0.0) if relu else acc


def _tc_layer_call(x, aggp, deg, w, wn, din, relu):
    grid_spec = pl.GridSpec(
        grid=(_GRID,),
        in_specs=[
            pl.BlockSpec((_R, din), lambda i: (i, 0)),
            pl.BlockSpec((1, _R, din), lambda i: (0, i, 0)),
            pl.BlockSpec((1, _R, din), lambda i: (1, i, 0)),
            pl.BlockSpec((_R, 1), lambda i: (i, 0)),
            pl.BlockSpec((din, DOUT), lambda i: (0, 0)),
            pl.BlockSpec((din, DOUT), lambda i: (0, 0)),
        ],
        out_specs=pl.BlockSpec((_R, DOUT), lambda i: (i, 0)),
    )
    return pl.pallas_call(
        functools.partial(_tc_layer, relu=relu),
        grid_spec=grid_spec,
        out_shape=jax.ShapeDtypeStruct((N, DOUT), jnp.float32),
    )(x, aggp, aggp, deg, w, wn)


def _tc_loss(emb_ref, out_ref):
    ctr = emb_ref[0:NGRP]
    pos = emb_ref[NGRP:2 * NGRP]
    n0 = emb_ref[2 * NGRP:3 * NGRP]
    n1 = emb_ref[3 * NGRP:4 * NGRP]
    n2 = emb_ref[4 * NGRP:5 * NGRP]
    pos_d = jnp.sum(ctr * pos, axis=1, keepdims=True)
    d0 = jnp.sum(ctr * n0, axis=1, keepdims=True)
    d1 = jnp.sum(ctr * n1, axis=1, keepdims=True)
    d2 = jnp.sum(ctr * n2, axis=1, keepdims=True)
    neg_d = jnp.maximum(jnp.maximum(d0, d1), d2)
    loss = jnp.sum(jnp.maximum(neg_d - pos_d + 1.0, 0.0)) * (1.0 / NGRP)
    out_ref[...] = jnp.reshape(loss, (1, 1))


def kernel(x, edge_index, seed_idx, batch_idx, Wp1s, Wp1n, Wp2s, Wp2n,
           Wt1s, Wt1n, Wt2s, Wt2n):
    f32 = jnp.float32
    agg_call, agg2_call, seed_call = _build_sc_calls()
    src = edge_index[0]
    dst = edge_index[1]
    src3 = jnp.pad(src.reshape(NW, EPW),
                   ((0, 0), (0, EPWP - EPW))).reshape(NW, NCH, CH)
    dst3 = jnp.pad(dst.reshape(NW, EPW), ((0, 0), (0, EPWP - EPW)),
                   constant_values=N).reshape(NW, NCH, CH)
    z128 = jnp.zeros((NPAD, DF), f32)
    (agg1p,) = agg_call(x, src3, dst3, z128)
    deg = _tc_deg_call(dst)

    pad = jnp.zeros((DF, DOUT - 40), f32)
    W1 = jnp.concatenate([Wp1s, Wt1s, pad], axis=1)
    W1n = jnp.concatenate([Wp1n, Wt1n, pad], axis=1)
    h = _tc_layer_call(x, agg1p, deg, W1, W1n, DF, True)

    (agg2p,) = agg2_call(h, src3, dst3, z128)

    W2 = jnp.zeros((DOUT, DOUT), f32)
    W2 = W2.at[0:32, 0:32].set(Wp2s).at[32:40, 32:40].set(Wt2s)
    W2n = jnp.zeros((DOUT, DOUT), f32)
    W2n = W2n.at[0:32, 0:32].set(Wp2n).at[32:40, 32:40].set(Wt2n)
    outn = _tc_layer_call(h, agg2p, deg, W2, W2n, DOUT, False)

    sp = seed_idx.reshape(NGRP, GROUP).T.reshape(-1)
    sp = jnp.concatenate([sp, jnp.zeros((NSEED - N,), jnp.int32)])
    (emb,) = seed_call(outn, sp.reshape(NW, 3, SCHUNK))

    loss = pl.pallas_call(
        _tc_loss,
        out_shape=jax.ShapeDtypeStruct((1, 1), jnp.float32),
    )(emb)
    return loss[0, 0]


# speedup vs baseline: 1.1808x; 1.1808x over previous
import jax
import jax.numpy as jnp
from jax import lax
from jax.experimental import pallas as pl
from jax.experimental.pallas import tpu as pltpu
from jax.experimental.pallas import tpu_sc as plsc

N = 10000
NPAD = 10240
E = 320000
DF = 128
DOUT = 128
DEGL = 40
GROUP = 5
NGRP = N // GROUP
NC = 2
NS = 16
NW = NC * NS
EPW = E // NW
CHUNK = 80
NCHUNK = EPW // CHUNK
RPW = NPAD // NS
SCHUNK = 128
SPW = 3 * SCHUNK
NSEED = NW * SPW
SPS = NSEED // NS

_R = 2000
_GRID = N // _R
_EB = 6400
_NEB = E // _EB


def _sc_agg(x_hbm, src_hbm, dst_hbm, z128_hbm, agg_out,
            src_v, dst_v, rows_v, agg_sh):
    c = lax.axis_index("c")
    s = lax.axis_index("s")
    w = s * NC + c
    r0 = s * RPW
    pltpu.sync_copy(z128_hbm.at[pl.ds(r0, RPW)], agg_sh.at[pl.ds(r0, RPW)])
    plsc.subcore_barrier()
    e0 = w * EPW

    def body(g, carry):
        base = e0 + g * CHUNK
        pltpu.sync_copy(src_hbm.at[pl.ds(base, CHUNK)], src_v)
        pltpu.sync_copy(dst_hbm.at[pl.ds(base, CHUNK)], dst_v)
        pltpu.sync_copy(x_hbm.at[src_v], rows_v)
        pltpu.sync_copy(rows_v, agg_sh.at[dst_v], add=True)
        return carry

    lax.fori_loop(0, NCHUNK, body, 0)
    plsc.subcore_barrier()
    pltpu.sync_copy(agg_sh.at[pl.ds(r0, RPW)],
                    agg_out.at[c, pl.ds(r0, RPW)])


def _sc_seed(h_hbm, aggf_hbm, sp_hbm, sp2_hbm, embh_out, emba_out,
             idx_v, grow_v):
    c = lax.axis_index("c")
    s = lax.axis_index("s")
    w = s * NC + c

    def body_h(j, carry):
        base = w * SPW + j * SCHUNK
        pltpu.sync_copy(sp_hbm.at[pl.ds(base, SCHUNK)], idx_v)
        pltpu.sync_copy(h_hbm.at[idx_v], grow_v)
        pltpu.sync_copy(grow_v, embh_out.at[pl.ds(base, SCHUNK)])
        return carry

    lax.fori_loop(0, SPW // SCHUNK, body_h, 0)

    def body_a(j, carry):
        base = s * SPS + j * SCHUNK
        pltpu.sync_copy(sp2_hbm.at[pl.ds(c * NSEED + base, SCHUNK)], idx_v)
        pltpu.sync_copy(aggf_hbm.at[idx_v], grow_v)
        pltpu.sync_copy(grow_v, emba_out.at[c, pl.ds(base, SCHUNK)])
        return carry

    lax.fori_loop(0, SPS // SCHUNK, body_a, 0)


_sc_calls = None


def _build_sc_calls():
    global _sc_calls
    if _sc_calls is not None:
        return _sc_calls
    mesh = plsc.VectorSubcoreMesh(core_axis_name="c", subcore_axis_name="s")
    agg_call = pl.kernel(
        _sc_agg, mesh=mesh,
        out_type=[jax.ShapeDtypeStruct((NC, NPAD, DF), jnp.float32)],
        scratch_types=[
            pltpu.VMEM((CHUNK,), jnp.int32),
            pltpu.VMEM((CHUNK,), jnp.int32),
            pltpu.VMEM((CHUNK, DF), jnp.float32),
            pltpu.VMEM_SHARED((NPAD, DF), jnp.float32),
        ])
    seed_call = pl.kernel(
        _sc_seed, mesh=mesh,
        out_type=[jax.ShapeDtypeStruct((NSEED, DOUT), jnp.float32),
                  jax.ShapeDtypeStruct((NC, NSEED, DOUT), jnp.float32)],
        scratch_types=[
            pltpu.VMEM((SCHUNK,), jnp.int32),
            pltpu.VMEM((SCHUNK, DOUT), jnp.float32),
        ])
    _sc_calls = (agg_call, seed_call)
    return _sc_calls


def _tc_deg(dst_ref, deg_ref):
    @pl.when(pl.program_id(0) == 0)
    def _():
        deg_ref[...] = jnp.zeros((128, 128), jnp.float32)

    d = dst_ref[...]
    lane = lax.broadcasted_iota(jnp.int32, (_EB, 128), 1)
    oh_hi = ((d >> 7) == lane).astype(jnp.float32)
    oh_lo = ((d & 127) == lane).astype(jnp.float32)
    deg_ref[...] += lax.dot_general(
        oh_hi, oh_lo, (((0,), (0,)), ((), ())),
        preferred_element_type=jnp.float32)


def _tc_deg_call(dst2):
    return pl.pallas_call(
        _tc_deg,
        grid=(_NEB,),
        in_specs=[pl.BlockSpec((_EB, 1), lambda i: (i, 0))],
        out_specs=pl.BlockSpec((128, 128), lambda i: (0, 0)),
        out_shape=jax.ShapeDtypeStruct((128, 128), jnp.float32),
    )(dst2)


def _tc_layer1(x_ref, agga_ref, aggb_ref, deg_ref,
               w_ref, wn_ref, wn2_ref, h_ref, y2_ref):
    deg = jnp.maximum(deg_ref[...], 1.0)
    agg = (agga_ref[0] + aggb_ref[0]) / deg
    h = jnp.maximum(
        jnp.dot(x_ref[...], w_ref[...], preferred_element_type=jnp.float32)
        + jnp.dot(agg, wn_ref[...], preferred_element_type=jnp.float32), 0.0)
    h_ref[...] = h
    lane = lax.broadcasted_iota(jnp.int32, (_R, DOUT), 1)
    y2_ref[...] = (jnp.dot(h, wn2_ref[...],
                           preferred_element_type=jnp.float32)
                   + (lane == DEGL).astype(jnp.float32))


def _tc_layer1_call(x, aggp, deg, w, wn, wn2):
    return pl.pallas_call(
        _tc_layer1,
        grid=(_GRID,),
        in_specs=[
            pl.BlockSpec((_R, DF), lambda i: (i, 0)),
            pl.BlockSpec((1, _R, DF), lambda i: (0, i, 0)),
            pl.BlockSpec((1, _R, DF), lambda i: (1, i, 0)),
            pl.BlockSpec((_R, 1), lambda i: (i, 0)),
            pl.BlockSpec((DF, DOUT), lambda i: (0, 0)),
            pl.BlockSpec((DF, DOUT), lambda i: (0, 0)),
            pl.BlockSpec((DOUT, DOUT), lambda i: (0, 0)),
        ],
        out_specs=[pl.BlockSpec((_R, DOUT), lambda i: (i, 0)),
                   pl.BlockSpec((_R, DOUT), lambda i: (i, 0))],
        out_shape=[jax.ShapeDtypeStruct((N, DOUT), jnp.float32),
                   jax.ShapeDtypeStruct((N, DOUT), jnp.float32)],
    )(x, aggp, aggp, deg, w, wn, wn2)


def _tc_loss(embh_ref, a0_ref, a1_ref, w2s_ref, out_ref):
    asum = a0_ref[0] + a1_ref[0]
    deg = jnp.maximum(asum[:, DEGL:DEGL + 1], 1.0)
    lane = lax.broadcasted_iota(jnp.int32, (NSEED, DOUT), 1)
    aggn = jnp.where(lane < DEGL, asum / deg, 0.0)
    emb = (jnp.dot(embh_ref[...], w2s_ref[...],
                   preferred_element_type=jnp.float32) + aggn)
    ctr = emb[0:NGRP]
    pos = emb[NGRP:2 * NGRP]
    n0 = emb[2 * NGRP:3 * NGRP]
    n1 = emb[3 * NGRP:4 * NGRP]
    n2 = emb[4 * NGRP:5 * NGRP]
    pos_d = jnp.sum(ctr * pos, axis=1, keepdims=True)
    d0 = jnp.sum(ctr * n0, axis=1, keepdims=True)
    d1 = jnp.sum(ctr * n1, axis=1, keepdims=True)
    d2 = jnp.sum(ctr * n2, axis=1, keepdims=True)
    neg_d = jnp.maximum(jnp.maximum(d0, d1), d2)
    loss = jnp.sum(jnp.maximum(neg_d - pos_d + 1.0, 0.0)) * (1.0 / NGRP)
    out_ref[...] = jnp.reshape(loss, (1, 1))


def _tc_loss_call(embh, emba, w2s):
    return pl.pallas_call(
        _tc_loss,
        grid=(1,),
        in_specs=[
            pl.BlockSpec((NSEED, DOUT), lambda i: (0, 0)),
            pl.BlockSpec((1, NSEED, DOUT), lambda i: (0, 0, 0)),
            pl.BlockSpec((1, NSEED, DOUT), lambda i: (1, 0, 0)),
            pl.BlockSpec((DOUT, DOUT), lambda i: (0, 0)),
        ],
        out_specs=pl.BlockSpec((1, 1), lambda i: (0, 0)),
        out_shape=jax.ShapeDtypeStruct((1, 1), jnp.float32),
    )(embh, emba, emba, w2s)


def kernel(x, edge_index, seed_idx, batch_idx, Wp1s, Wp1n, Wp2s, Wp2n,
           Wt1s, Wt1n, Wt2s, Wt2n):
    f32 = jnp.float32
    agg_call, seed_call = _build_sc_calls()
    src = edge_index[0]
    dst = edge_index[1]
    z128 = jnp.zeros((NPAD, DF), f32)
    (agg1p,) = agg_call(x, src, dst, z128)

    deg2d = _tc_deg_call(dst.reshape(E, 1))
    deg = deg2d.reshape(128 * 128, 1)[0:N]

    pad = jnp.zeros((DF, DOUT - 40), f32)
    W1 = jnp.concatenate([Wp1s, Wt1s, pad], axis=1)
    W1n = jnp.concatenate([Wp1n, Wt1n, pad], axis=1)
    W2n_blk = jnp.zeros((DOUT, DOUT), f32)
    W2n_blk = W2n_blk.at[0:32, 0:32].set(Wp2n).at[32:40, 32:40].set(Wt2n)
    h, y2 = _tc_layer1_call(x, agg1p, deg, W1, W1n, W2n_blk)

    (aggp2,) = agg_call(y2, src, dst, z128)

    sp = seed_idx.reshape(NGRP, GROUP).T.reshape(-1)
    sp = jnp.concatenate([sp, jnp.zeros((NSEED - N,), jnp.int32)])
    sp2 = jnp.concatenate([sp, sp + NPAD])
    embh, emba = seed_call(h, aggp2.reshape(NC * NPAD, DOUT), sp, sp2)

    W2s_blk = jnp.zeros((DOUT, DOUT), f32)
    W2s_blk = W2s_blk.at[0:32, 0:32].set(Wp2s).at[32:40, 32:40].set(Wt2s)
    loss = _tc_loss_call(embh, emba, W2s_blk)
    return loss[0, 0]


# speedup vs baseline: 1.3105x; 1.1098x over previous
import functools

import jax
import jax.numpy as jnp
from jax import lax
from jax.experimental import pallas as pl
from jax.experimental.pallas import tpu as pltpu
from jax.experimental.pallas import tpu_sc as plsc

N = 10000
NPAD = 10240
E = 320000
DF = 128
DOUT = 128
GROUP = 5
NGRP = N // GROUP
NC = 2
NS = 16
NW = NC * NS
EPW = E // NW
CHUNK = 80
NCHUNK = EPW // CHUNK
RPW = NPAD // NS
SCHUNK = 128
SPW = 3 * SCHUNK
NSEED = NW * SPW

_R = 2000
_GRID = N // _R
_EB = 6400
_NEB = E // _EB


def _sc_agg(x_hbm, src_hbm, dst_hbm, z128_hbm, agg_out,
            src_v, dst_v, rows_v, agg_sh):
    c = lax.axis_index("c")
    s = lax.axis_index("s")
    w = s * NC + c
    r0 = s * RPW
    pltpu.sync_copy(z128_hbm.at[pl.ds(r0, RPW)], agg_sh.at[pl.ds(r0, RPW)])
    plsc.subcore_barrier()
    e0 = w * EPW

    def body(g, carry):
        base = e0 + g * CHUNK
        pltpu.sync_copy(src_hbm.at[pl.ds(base, CHUNK)], src_v)
        pltpu.sync_copy(dst_hbm.at[pl.ds(base, CHUNK)], dst_v)
        pltpu.sync_copy(x_hbm.at[src_v], rows_v)
        pltpu.sync_copy(rows_v, agg_sh.at[dst_v], add=True)
        return carry

    lax.fori_loop(0, NCHUNK, body, 0)
    plsc.subcore_barrier()
    pltpu.sync_copy(agg_sh.at[pl.ds(r0, RPW)],
                    agg_out.at[c, pl.ds(r0, RPW)])


def _sc_seed_gather(out_hbm, seed_hbm, emb_out, idx_v, rows_v):
    c = lax.axis_index("c")
    s = lax.axis_index("s")
    w = s * NC + c

    def body(j, carry):
        base = w * SPW + j * SCHUNK
        pltpu.sync_copy(seed_hbm.at[pl.ds(base, SCHUNK)], idx_v)
        pltpu.sync_copy(out_hbm.at[idx_v], rows_v)
        pltpu.sync_copy(rows_v, emb_out.at[pl.ds(base, SCHUNK)])
        return carry

    lax.fori_loop(0, SPW // SCHUNK, body, 0)


_sc_calls = None


def _build_sc_calls():
    global _sc_calls
    if _sc_calls is not None:
        return _sc_calls
    mesh = plsc.VectorSubcoreMesh(core_axis_name="c", subcore_axis_name="s")
    agg_call = pl.kernel(
        _sc_agg, mesh=mesh,
        out_type=[jax.ShapeDtypeStruct((NC, NPAD, DF), jnp.float32)],
        scratch_types=[
            pltpu.VMEM((CHUNK,), jnp.int32),
            pltpu.VMEM((CHUNK,), jnp.int32),
            pltpu.VMEM((CHUNK, DF), jnp.float32),
            pltpu.VMEM_SHARED((NPAD, DF), jnp.float32),
        ])
    seed_call = pl.kernel(
        _sc_seed_gather, mesh=mesh,
        out_type=[jax.ShapeDtypeStruct((NSEED, DOUT), jnp.float32)],
        scratch_types=[
            pltpu.VMEM((SCHUNK,), jnp.int32),
            pltpu.VMEM((SCHUNK, DOUT), jnp.float32),
        ])
    _sc_calls = (agg_call, seed_call)
    return _sc_calls


def _tc_deg(dst_ref, deg_ref):
    @pl.when(pl.program_id(0) == 0)
    def _():
        deg_ref[...] = jnp.zeros((128, 128), jnp.float32)

    d = dst_ref[...]
    lane = lax.broadcasted_iota(jnp.int32, (_EB, 128), 1)
    oh_hi = ((d >> 7) == lane).astype(jnp.float32)
    oh_lo = ((d & 127) == lane).astype(jnp.float32)
    deg_ref[...] += lax.dot_general(
        oh_hi, oh_lo, (((0,), (0,)), ((), ())),
        preferred_element_type=jnp.float32)


def _tc_deg_call(dst2):
    return pl.pallas_call(
        _tc_deg,
        grid=(_NEB,),
        in_specs=[pl.BlockSpec((_EB, 1), lambda i: (i, 0))],
        out_specs=pl.BlockSpec((128, 128), lambda i: (0, 0)),
        out_shape=jax.ShapeDtypeStruct((128, 128), jnp.float32),
    )(dst2)


def _tc_layer(x_ref, agga_ref, aggb_ref, deg_ref, w_ref, wn_ref,
              h_ref, *, relu):
    deg = jnp.maximum(deg_ref[...], 1.0)
    agg = (agga_ref[0] + aggb_ref[0]) / deg
    acc = (jnp.dot(x_ref[...], w_ref[...], preferred_element_type=jnp.float32)
           + jnp.dot(agg, wn_ref[...], preferred_element_type=jnp.float32))
    h_ref[...] = jnp.maximum(acc, 0.0) if relu else acc


def _tc_layer_call(x, aggp, deg, w, wn, din, relu):
    return pl.pallas_call(
        functools.partial(_tc_layer, relu=relu),
        grid=(_GRID,),
        in_specs=[
            pl.BlockSpec((_R, din), lambda i: (i, 0)),
            pl.BlockSpec((1, _R, din), lambda i: (0, i, 0)),
            pl.BlockSpec((1, _R, din), lambda i: (1, i, 0)),
            pl.BlockSpec((_R, 1), lambda i: (i, 0)),
            pl.BlockSpec((din, DOUT), lambda i: (0, 0)),
            pl.BlockSpec((din, DOUT), lambda i: (0, 0)),
        ],
        out_specs=pl.BlockSpec((_R, DOUT), lambda i: (i, 0)),
        out_shape=jax.ShapeDtypeStruct((N, DOUT), jnp.float32),
    )(x, aggp, aggp, deg, w, wn)


def _tc_loss(emb_ref, out_ref):
    ctr = emb_ref[0:NGRP]
    pos = emb_ref[NGRP:2 * NGRP]
    n0 = emb_ref[2 * NGRP:3 * NGRP]
    n1 = emb_ref[3 * NGRP:4 * NGRP]
    n2 = emb_ref[4 * NGRP:5 * NGRP]
    pos_d = jnp.sum(ctr * pos, axis=1, keepdims=True)
    d0 = jnp.sum(ctr * n0, axis=1, keepdims=True)
    d1 = jnp.sum(ctr * n1, axis=1, keepdims=True)
    d2 = jnp.sum(ctr * n2, axis=1, keepdims=True)
    neg_d = jnp.maximum(jnp.maximum(d0, d1), d2)
    loss = jnp.sum(jnp.maximum(neg_d - pos_d + 1.0, 0.0)) * (1.0 / NGRP)
    out_ref[...] = jnp.reshape(loss, (1, 1))


def kernel(x, edge_index, seed_idx, batch_idx, Wp1s, Wp1n, Wp2s, Wp2n,
           Wt1s, Wt1n, Wt2s, Wt2n):
    f32 = jnp.float32
    agg_call, seed_call = _build_sc_calls()
    src = edge_index[0]
    dst = edge_index[1]
    z128 = jnp.zeros((NPAD, DF), f32)
    (agg1p,) = agg_call(x, src, dst, z128)

    deg2d = _tc_deg_call(dst.reshape(E, 1))
    deg = deg2d.reshape(128 * 128, 1)[0:N]

    pad = jnp.zeros((DF, DOUT - 40), f32)
    W1 = jnp.concatenate([Wp1s, Wt1s, pad], axis=1)
    W1n = jnp.concatenate([Wp1n, Wt1n, pad], axis=1)
    h = _tc_layer_call(x, agg1p, deg, W1, W1n, DF, True)

    (agg2p,) = agg_call(h, src, dst, z128)

    W2 = jnp.zeros((DOUT, DOUT), f32)
    W2 = W2.at[0:32, 0:32].set(Wp2s).at[32:40, 32:40].set(Wt2s)
    W2n = jnp.zeros((DOUT, DOUT), f32)
    W2n = W2n.at[0:32, 0:32].set(Wp2n).at[32:40, 32:40].set(Wt2n)
    outn = _tc_layer_call(h, agg2p, deg, W2, W2n, DOUT, False)

    sp = seed_idx.reshape(NGRP, GROUP).T.reshape(-1)
    sp = jnp.concatenate([sp, jnp.zeros((NSEED - N,), jnp.int32)])
    (emb,) = seed_call(outn, sp)

    loss = pl.pallas_call(
        _tc_loss,
        out_shape=jax.ShapeDtypeStruct((1, 1), jnp.float32),
    )(emb)
    return loss[0, 0]


# speedup vs baseline: 1.8794x; 1.4342x over previous
import functools

import jax
import jax.numpy as jnp
from jax import lax
from jax.experimental import pallas as pl
from jax.experimental.pallas import tpu as pltpu
from jax.experimental.pallas import tpu_sc as plsc

N = 10000
NPAD = 10240
E = 320000
DF = 128
DOUT = 128
GROUP = 5
NGRP = N // GROUP
NC = 2
NS = 16
NW = NC * NS
EPW = E // NW
CHUNK = 80
NCHUNK = EPW // CHUNK
RPW = NPAD // NS
SCHUNK = 128
SPW = 3 * SCHUNK
NSEED = NW * SPW

_R = 2000
_GRID = N // _R
_EB = 6400
_NEB = E // _EB


def _sc_agg(x_hbm, src_hbm, dst_hbm, z128_hbm, agg_out,
            src0_v, src1_v, dst0_v, dst1_v, rows0_v, rows1_v,
            sem0, sem1, agg_sh):
    c = lax.axis_index("c")
    s = lax.axis_index("s")
    w = s * NC + c
    r0 = s * RPW
    pltpu.sync_copy(z128_hbm.at[pl.ds(r0, RPW)], agg_sh.at[pl.ds(r0, RPW)])
    plsc.subcore_barrier()
    e0 = w * EPW
    src_v = (src0_v, src1_v)
    dst_v = (dst0_v, dst1_v)
    rows_v = (rows0_v, rows1_v)
    sems = (sem0, sem1)

    pltpu.sync_copy(src_hbm.at[pl.ds(e0, CHUNK)], src0_v)
    pltpu.sync_copy(dst_hbm.at[pl.ds(e0, CHUNK)], dst0_v)
    pltpu.async_copy(x_hbm.at[src0_v], rows0_v, sem0)

    def body(gg, carry):
        for b in range(2):
            g = gg * 2 + b
            nb = 1 - b
            nbase = e0 + (g + 1) * CHUNK
            pltpu.sync_copy(src_hbm.at[pl.ds(nbase, CHUNK)], src_v[nb])
            pltpu.sync_copy(dst_hbm.at[pl.ds(nbase, CHUNK)], dst_v[nb])
            pltpu.async_copy(x_hbm.at[src_v[nb]], rows_v[nb], sems[nb])
            pltpu.make_async_copy(x_hbm.at[src_v[b]], rows_v[b],
                                  sems[b]).wait()
            pltpu.sync_copy(rows_v[b], agg_sh.at[dst_v[b]], add=True)
        return carry

    lax.fori_loop(0, (NCHUNK - 1) // 2, body, 0)
    lb = (NCHUNK - 1) % 2
    pltpu.make_async_copy(x_hbm.at[src_v[lb]], rows_v[lb], sems[lb]).wait()
    pltpu.sync_copy(rows_v[lb], agg_sh.at[dst_v[lb]], add=True)
    plsc.subcore_barrier()
    pltpu.sync_copy(agg_sh.at[pl.ds(r0, RPW)],
                    agg_out.at[c, pl.ds(r0, RPW)])


def _sc_seed_gather(out_hbm, seed_hbm, emb_out, idx_v, rows_v):
    c = lax.axis_index("c")
    s = lax.axis_index("s")
    w = s * NC + c

    def body(j, carry):
        base = w * SPW + j * SCHUNK
        pltpu.sync_copy(seed_hbm.at[pl.ds(base, SCHUNK)], idx_v)
        pltpu.sync_copy(out_hbm.at[idx_v], rows_v)
        pltpu.sync_copy(rows_v, emb_out.at[pl.ds(base, SCHUNK)])
        return carry

    lax.fori_loop(0, SPW // SCHUNK, body, 0)


_sc_calls = None


def _build_sc_calls():
    global _sc_calls
    if _sc_calls is not None:
        return _sc_calls
    mesh = plsc.VectorSubcoreMesh(core_axis_name="c", subcore_axis_name="s")
    agg_call = pl.kernel(
        _sc_agg, mesh=mesh,
        out_type=[jax.ShapeDtypeStruct((NC, NPAD, DF), jnp.float32)],
        scratch_types=[
            pltpu.VMEM((CHUNK,), jnp.int32),
            pltpu.VMEM((CHUNK,), jnp.int32),
            pltpu.VMEM((CHUNK,), jnp.int32),
            pltpu.VMEM((CHUNK,), jnp.int32),
            pltpu.VMEM((CHUNK, DF), jnp.float32),
            pltpu.VMEM((CHUNK, DF), jnp.float32),
            pltpu.SemaphoreType.DMA,
            pltpu.SemaphoreType.DMA,
            pltpu.VMEM_SHARED((NPAD, DF), jnp.float32),
        ])
    seed_call = pl.kernel(
        _sc_seed_gather, mesh=mesh,
        out_type=[jax.ShapeDtypeStruct((NSEED, DOUT), jnp.float32)],
        scratch_types=[
            pltpu.VMEM((SCHUNK,), jnp.int32),
            pltpu.VMEM((SCHUNK, DOUT), jnp.float32),
        ])
    _sc_calls = (agg_call, seed_call)
    return _sc_calls


def _tc_deg(dst_ref, deg_ref):
    @pl.when(pl.program_id(0) == 0)
    def _():
        deg_ref[...] = jnp.zeros((128, 128), jnp.float32)

    d = dst_ref[...]
    lane = lax.broadcasted_iota(jnp.int32, (_EB, 128), 1)
    oh_hi = ((d >> 7) == lane).astype(jnp.float32)
    oh_lo = ((d & 127) == lane).astype(jnp.float32)
    deg_ref[...] += lax.dot_general(
        oh_hi, oh_lo, (((0,), (0,)), ((), ())),
        preferred_element_type=jnp.float32)


def _tc_deg_call(dst2):
    return pl.pallas_call(
        _tc_deg,
        grid=(_NEB,),
        in_specs=[pl.BlockSpec((_EB, 1), lambda i: (i, 0))],
        out_specs=pl.BlockSpec((128, 128), lambda i: (0, 0)),
        out_shape=jax.ShapeDtypeStruct((128, 128), jnp.float32),
    )(dst2)


def _tc_layer(x_ref, agga_ref, aggb_ref, deg_ref, w_ref, wn_ref,
              h_ref, *, relu):
    deg = jnp.maximum(deg_ref[...], 1.0)
    agg = (agga_ref[0] + aggb_ref[0]) / deg
    acc = (jnp.dot(x_ref[...], w_ref[...], preferred_element_type=jnp.float32)
           + jnp.dot(agg, wn_ref[...], preferred_element_type=jnp.float32))
    h_ref[...] = jnp.maximum(acc, 0.0) if relu else acc


def _tc_layer_call(x, aggp, deg, w, wn, din, relu):
    return pl.pallas_call(
        functools.partial(_tc_layer, relu=relu),
        grid=(_GRID,),
        in_specs=[
            pl.BlockSpec((_R, din), lambda i: (i, 0)),
            pl.BlockSpec((1, _R, din), lambda i: (0, i, 0)),
            pl.BlockSpec((1, _R, din), lambda i: (1, i, 0)),
            pl.BlockSpec((_R, 1), lambda i: (i, 0)),
            pl.BlockSpec((din, DOUT), lambda i: (0, 0)),
            pl.BlockSpec((din, DOUT), lambda i: (0, 0)),
        ],
        out_specs=pl.BlockSpec((_R, DOUT), lambda i: (i, 0)),
        out_shape=jax.ShapeDtypeStruct((N, DOUT), jnp.float32),
    )(x, aggp, aggp, deg, w, wn)


def _tc_loss(emb_ref, out_ref):
    ctr = emb_ref[0:NGRP]
    pos = emb_ref[NGRP:2 * NGRP]
    n0 = emb_ref[2 * NGRP:3 * NGRP]
    n1 = emb_ref[3 * NGRP:4 * NGRP]
    n2 = emb_ref[4 * NGRP:5 * NGRP]
    pos_d = jnp.sum(ctr * pos, axis=1, keepdims=True)
    d0 = jnp.sum(ctr * n0, axis=1, keepdims=True)
    d1 = jnp.sum(ctr * n1, axis=1, keepdims=True)
    d2 = jnp.sum(ctr * n2, axis=1, keepdims=True)
    neg_d = jnp.maximum(jnp.maximum(d0, d1), d2)
    loss = jnp.sum(jnp.maximum(neg_d - pos_d + 1.0, 0.0)) * (1.0 / NGRP)
    out_ref[...] = jnp.reshape(loss, (1, 1))


def kernel(x, edge_index, seed_idx, batch_idx, Wp1s, Wp1n, Wp2s, Wp2n,
           Wt1s, Wt1n, Wt2s, Wt2n):
    f32 = jnp.float32
    agg_call, seed_call = _build_sc_calls()
    src = edge_index[0]
    dst = edge_index[1]
    z128 = jnp.zeros((NPAD, DF), f32)
    (agg1p,) = agg_call(x, src, dst, z128)

    deg2d = _tc_deg_call(dst.reshape(E, 1))
    deg = deg2d.reshape(128 * 128, 1)[0:N]

    pad = jnp.zeros((DF, DOUT - 40), f32)
    W1 = jnp.concatenate([Wp1s, Wt1s, pad], axis=1)
    W1n = jnp.concatenate([Wp1n, Wt1n, pad], axis=1)
    h = _tc_layer_call(x, agg1p, deg, W1, W1n, DF, True)

    (agg2p,) = agg_call(h, src, dst, z128)

    W2 = jnp.zeros((DOUT, DOUT), f32)
    W2 = W2.at[0:32, 0:32].set(Wp2s).at[32:40, 32:40].set(Wt2s)
    W2n = jnp.zeros((DOUT, DOUT), f32)
    W2n = W2n.at[0:32, 0:32].set(Wp2n).at[32:40, 32:40].set(Wt2n)
    outn = _tc_layer_call(h, agg2p, deg, W2, W2n, DOUT, False)

    sp = seed_idx.reshape(NGRP, GROUP).T.reshape(-1)
    sp = jnp.concatenate([sp, jnp.zeros((NSEED - N,), jnp.int32)])
    (emb,) = seed_call(outn, sp)

    loss = pl.pallas_call(
        _tc_loss,
        out_shape=jax.ShapeDtypeStruct((1, 1), jnp.float32),
    )(emb)
    return loss[0, 0]


# speedup vs baseline: 1.8838x; 1.0023x over previous
import functools

import jax
import jax.numpy as jnp
from jax import lax
from jax.experimental import pallas as pl
from jax.experimental.pallas import tpu as pltpu
from jax.experimental.pallas import tpu_sc as plsc

N = 10000
NPAD = 10240
E = 320000
DF = 128
DOUT = 128
GROUP = 5
NGRP = N // GROUP
NC = 2
NS = 16
NW = NC * NS
EPW = E // NW
CHUNK = 80
NCHUNK = EPW // CHUNK
RPW = NPAD // NS
SCHUNK = 128
SPW = 3 * SCHUNK
NSEED = NW * SPW

_R = 2000
_GRID = N // _R
_EB = 6400
_NEB = E // _EB


def _sc_agg(x_hbm, src_hbm, dst_hbm, z128_hbm, agg_out,
            src0_v, src1_v, dst0_v, dst1_v, rows0_v, rows1_v,
            sem0, sem1, agg_sh):
    c = lax.axis_index("c")
    s = lax.axis_index("s")
    w = s * NC + c
    r0 = s * RPW
    pltpu.sync_copy(z128_hbm.at[pl.ds(r0, RPW)], agg_sh.at[pl.ds(r0, RPW)])
    plsc.subcore_barrier()
    e0 = w * EPW
    src_v = (src0_v, src1_v)
    dst_v = (dst0_v, dst1_v)
    rows_v = (rows0_v, rows1_v)
    sems = (sem0, sem1)

    pltpu.sync_copy(src_hbm.at[pl.ds(e0, CHUNK)], src0_v)
    pltpu.sync_copy(dst_hbm.at[pl.ds(e0, CHUNK)], dst0_v)
    pltpu.async_copy(x_hbm.at[src0_v], rows0_v, sem0)

    def body(gg, carry):
        for b in range(2):
            g = gg * 2 + b
            nb = 1 - b
            nbase = e0 + (g + 1) * CHUNK
            pltpu.sync_copy(src_hbm.at[pl.ds(nbase, CHUNK)], src_v[nb])
            pltpu.sync_copy(dst_hbm.at[pl.ds(nbase, CHUNK)], dst_v[nb])
            pltpu.async_copy(x_hbm.at[src_v[nb]], rows_v[nb], sems[nb])
            pltpu.make_async_copy(x_hbm.at[src_v[b]], rows_v[b],
                                  sems[b]).wait()
            pltpu.sync_copy(rows_v[b], agg_sh.at[dst_v[b]], add=True)
        return carry

    lax.fori_loop(0, (NCHUNK - 1) // 2, body, 0)
    lb = (NCHUNK - 1) % 2
    pltpu.make_async_copy(x_hbm.at[src_v[lb]], rows_v[lb], sems[lb]).wait()
    pltpu.sync_copy(rows_v[lb], agg_sh.at[dst_v[lb]], add=True)
    plsc.subcore_barrier()
    pltpu.sync_copy(agg_sh.at[pl.ds(r0, RPW)],
                    agg_out.at[c, pl.ds(r0, RPW)])


def _sc_seed_gather(out_hbm, seed_hbm, emb_out, idx0_v, idx1_v,
                    rows0_v, rows1_v, sem0, sem1):
    c = lax.axis_index("c")
    s = lax.axis_index("s")
    w = s * NC + c
    b0 = w * SPW
    pltpu.sync_copy(seed_hbm.at[pl.ds(b0, SCHUNK)], idx0_v)
    pltpu.async_copy(out_hbm.at[idx0_v], rows0_v, sem0)
    pltpu.sync_copy(seed_hbm.at[pl.ds(b0 + SCHUNK, SCHUNK)], idx1_v)
    pltpu.async_copy(out_hbm.at[idx1_v], rows1_v, sem1)
    pltpu.make_async_copy(out_hbm.at[idx0_v], rows0_v, sem0).wait()
    pltpu.sync_copy(rows0_v, emb_out.at[pl.ds(b0, SCHUNK)])
    pltpu.sync_copy(seed_hbm.at[pl.ds(b0 + 2 * SCHUNK, SCHUNK)], idx0_v)
    pltpu.async_copy(out_hbm.at[idx0_v], rows0_v, sem0)
    pltpu.make_async_copy(out_hbm.at[idx1_v], rows1_v, sem1).wait()
    pltpu.sync_copy(rows1_v, emb_out.at[pl.ds(b0 + SCHUNK, SCHUNK)])
    pltpu.make_async_copy(out_hbm.at[idx0_v], rows0_v, sem0).wait()
    pltpu.sync_copy(rows0_v, emb_out.at[pl.ds(b0 + 2 * SCHUNK, SCHUNK)])


_sc_calls = None


def _build_sc_calls():
    global _sc_calls
    if _sc_calls is not None:
        return _sc_calls
    mesh = plsc.VectorSubcoreMesh(core_axis_name="c", subcore_axis_name="s")
    agg_call = pl.kernel(
        _sc_agg, mesh=mesh,
        out_type=[jax.ShapeDtypeStruct((NC, NPAD, DF), jnp.float32)],
        scratch_types=[
            pltpu.VMEM((CHUNK,), jnp.int32),
            pltpu.VMEM((CHUNK,), jnp.int32),
            pltpu.VMEM((CHUNK,), jnp.int32),
            pltpu.VMEM((CHUNK,), jnp.int32),
            pltpu.VMEM((CHUNK, DF), jnp.float32),
            pltpu.VMEM((CHUNK, DF), jnp.float32),
            pltpu.SemaphoreType.DMA,
            pltpu.SemaphoreType.DMA,
            pltpu.VMEM_SHARED((NPAD, DF), jnp.float32),
        ])
    seed_call = pl.kernel(
        _sc_seed_gather, mesh=mesh,
        out_type=[jax.ShapeDtypeStruct((NSEED, DOUT), jnp.float32)],
        scratch_types=[
            pltpu.VMEM((SCHUNK,), jnp.int32),
            pltpu.VMEM((SCHUNK,), jnp.int32),
            pltpu.VMEM((SCHUNK, DOUT), jnp.float32),
            pltpu.VMEM((SCHUNK, DOUT), jnp.float32),
            pltpu.SemaphoreType.DMA,
            pltpu.SemaphoreType.DMA,
        ])
    _sc_calls = (agg_call, seed_call)
    return _sc_calls


def _tc_deg(dst_ref, deg_ref):
    @pl.when(pl.program_id(0) == 0)
    def _():
        deg_ref[...] = jnp.zeros((128, 128), jnp.float32)

    d = dst_ref[...]
    lane = lax.broadcasted_iota(jnp.int32, (_EB, 128), 1)
    oh_hi = ((d >> 7) == lane).astype(jnp.float32)
    oh_lo = ((d & 127) == lane).astype(jnp.float32)
    deg_ref[...] += lax.dot_general(
        oh_hi, oh_lo, (((0,), (0,)), ((), ())),
        preferred_element_type=jnp.float32)


def _tc_deg_call(dst2):
    return pl.pallas_call(
        _tc_deg,
        grid=(_NEB,),
        in_specs=[pl.BlockSpec((_EB, 1), lambda i: (i, 0))],
        out_specs=pl.BlockSpec((128, 128), lambda i: (0, 0)),
        out_shape=jax.ShapeDtypeStruct((128, 128), jnp.float32),
    )(dst2)


def _tc_layer(x_ref, agga_ref, aggb_ref, deg_ref, w_ref, wn_ref,
              h_ref, *, relu):
    deg = jnp.maximum(deg_ref[...], 1.0)
    agg = (agga_ref[0] + aggb_ref[0]) / deg
    acc = (jnp.dot(x_ref[...], w_ref[...], preferred_element_type=jnp.float32)
           + jnp.dot(agg, wn_ref[...], preferred_element_type=jnp.float32))
    h_ref[...] = jnp.maximum(acc, 0.0) if relu else acc


def _tc_layer_call(x, aggp, deg, w, wn, din, relu):
    return pl.pallas_call(
        functools.partial(_tc_layer, relu=relu),
        grid=(_GRID,),
        in_specs=[
            pl.BlockSpec((_R, din), lambda i: (i, 0)),
            pl.BlockSpec((1, _R, din), lambda i: (0, i, 0)),
            pl.BlockSpec((1, _R, din), lambda i: (1, i, 0)),
            pl.BlockSpec((_R, 1), lambda i: (i, 0)),
            pl.BlockSpec((din, DOUT), lambda i: (0, 0)),
            pl.BlockSpec((din, DOUT), lambda i: (0, 0)),
        ],
        out_specs=pl.BlockSpec((_R, DOUT), lambda i: (i, 0)),
        out_shape=jax.ShapeDtypeStruct((N, DOUT), jnp.float32),
    )(x, aggp, aggp, deg, w, wn)


def _tc_loss(emb_ref, out_ref):
    ctr = emb_ref[0:NGRP]
    pos = emb_ref[NGRP:2 * NGRP]
    n0 = emb_ref[2 * NGRP:3 * NGRP]
    n1 = emb_ref[3 * NGRP:4 * NGRP]
    n2 = emb_ref[4 * NGRP:5 * NGRP]
    pos_d = jnp.sum(ctr * pos, axis=1, keepdims=True)
    d0 = jnp.sum(ctr * n0, axis=1, keepdims=True)
    d1 = jnp.sum(ctr * n1, axis=1, keepdims=True)
    d2 = jnp.sum(ctr * n2, axis=1, keepdims=True)
    neg_d = jnp.maximum(jnp.maximum(d0, d1), d2)
    loss = jnp.sum(jnp.maximum(neg_d - pos_d + 1.0, 0.0)) * (1.0 / NGRP)
    out_ref[...] = jnp.reshape(loss, (1, 1))


def kernel(x, edge_index, seed_idx, batch_idx, Wp1s, Wp1n, Wp2s, Wp2n,
           Wt1s, Wt1n, Wt2s, Wt2n):
    f32 = jnp.float32
    agg_call, seed_call = _build_sc_calls()
    src = edge_index[0]
    dst = edge_index[1]
    z128 = jnp.zeros((NPAD, DF), f32)
    (agg1p,) = agg_call(x, src, dst, z128)

    deg2d = _tc_deg_call(dst.reshape(E, 1))
    deg = deg2d.reshape(128 * 128, 1)[0:N]

    pad = jnp.zeros((DF, DOUT - 40), f32)
    W1 = jnp.concatenate([Wp1s, Wt1s, pad], axis=1)
    W1n = jnp.concatenate([Wp1n, Wt1n, pad], axis=1)
    h = _tc_layer_call(x, agg1p, deg, W1, W1n, DF, True)

    (agg2p,) = agg_call(h, src, dst, z128)

    W2 = jnp.zeros((DOUT, DOUT), f32)
    W2 = W2.at[0:32, 0:32].set(Wp2s).at[32:40, 32:40].set(Wt2s)
    W2n = jnp.zeros((DOUT, DOUT), f32)
    W2n = W2n.at[0:32, 0:32].set(Wp2n).at[32:40, 32:40].set(Wt2n)
    outn = _tc_layer_call(h, agg2p, deg, W2, W2n, DOUT, False)

    sp = seed_idx.reshape(NGRP, GROUP).T.reshape(-1)
    sp = jnp.concatenate([sp, jnp.zeros((NSEED - N,), jnp.int32)])
    (emb,) = seed_call(outn, sp)

    loss = pl.pallas_call(
        _tc_loss,
        out_shape=jax.ShapeDtypeStruct((1, 1), jnp.float32),
    )(emb)
    return loss[0, 0]
